# baseline (speedup 1.0000x reference)
import functools

import jax
import jax.numpy as jnp
from jax import lax
from jax.experimental import pallas as pl
from jax.experimental.pallas import tpu as pltpu
from jax.experimental.pallas import tpu_sc as plsc

_N_ROWS = 4096
_N_COLS = 100000
_NC = 2
_NS = 16
_L = 16
_NW = _NC * _NS
_ROWS_PER_TILE = _N_ROWS // _NW
_CHUNKS = _ROWS_PER_TILE // _L


def _ganloss_body(prob_hbm, tgt_hbm, rew_hbm, out_hbm,
                  tgt_v, rew_v, idx_v, vals_v, buf_v, sums_v, shared, sem):
    cid = lax.axis_index("c")
    sid = lax.axis_index("s")
    wid = sid * _NC + cid
    base = wid * _ROWS_PER_TILE

    pltpu.sync_copy(tgt_hbm.at[pl.ds(base, _ROWS_PER_TILE)], tgt_v)
    pltpu.sync_copy(rew_hbm.at[pl.ds(base, _ROWS_PER_TILE)], rew_v)

    lanes = lax.iota(jnp.int32, _L)
    for j in range(_CHUNKS):
        t = tgt_v[pl.ds(j * _L, _L)]
        rows = (base + j * _L) + lanes
        idx_v[pl.ds(j * _L, _L)] = rows * jnp.int32(_N_COLS) + t

    pltpu.async_copy(prob_hbm.at[idx_v], vals_v, sem).wait()

    acc = jnp.zeros((_L,), jnp.float32)
    for j in range(_CHUNKS):
        acc = acc + vals_v[pl.ds(j * _L, _L)] * rew_v[pl.ds(j * _L, _L)]

    buf_v[...] = acc
    pltpu.sync_copy(buf_v, shared.at[pl.ds((cid * _NS + sid) * _L, _L)])
    plsc.subcore_barrier()

    @pl.when(sid == 0)
    def _():
        pltpu.sync_copy(shared.at[pl.ds(cid * _NS * _L, _NS * _L)], sums_v)
        tot = jnp.zeros((_L,), jnp.float32)
        for r in range(_NS):
            tot = tot + sums_v[pl.ds(r * _L, _L)]
        buf_v[...] = -tot
        pltpu.sync_copy(buf_v, out_hbm.at[pl.ds(cid * _L, _L)])


@jax.jit
def _ganloss(prob_flat, target_i32, reward):
    mesh = plsc.VectorSubcoreMesh(core_axis_name="c", subcore_axis_name="s")
    run = pl.kernel(
        _ganloss_body,
        out_type=jax.ShapeDtypeStruct((_NC * _L,), jnp.float32),
        mesh=mesh,
        scratch_types=[
            pltpu.VMEM((_ROWS_PER_TILE,), jnp.int32),
            pltpu.VMEM((_ROWS_PER_TILE,), jnp.float32),
            pltpu.VMEM((_ROWS_PER_TILE,), jnp.int32),
            pltpu.VMEM((_ROWS_PER_TILE,), jnp.float32),
            pltpu.VMEM((_L,), jnp.float32),
            pltpu.VMEM((_NS * _L,), jnp.float32),
            pltpu.VMEM_SHARED((_NW * _L,), jnp.float32),
            pltpu.SemaphoreType.DMA,
        ],
    )
    out = run(prob_flat, target_i32, reward)
    return jnp.sum(out)


def kernel(prob, target, reward):
    return _ganloss(prob.reshape(-1), target.astype(jnp.int32), reward)


# speedup vs baseline: 122.2909x; 122.2909x over previous
import jax
import jax.numpy as jnp
from jax import lax
from jax.experimental import pallas as pl
from jax.experimental.pallas import tpu as pltpu
from jax.experimental.pallas import tpu_sc as plsc

_N_ROWS = 4096
_N_COLS = 100000
_NC = 2
_NS = 16
_L = 16
_NW = _NC * _NS
_ROWS_PER_TILE = _N_ROWS // _NW
_CHUNKS = _ROWS_PER_TILE // _L


def _ganloss_body(pt_hbm, tgt_hbm, rew_hbm, out_hbm,
                  tgt_v, rew_v, vals_v, buf_v, sums_v, shared, sem):
    cid = lax.axis_index("c")
    sid = lax.axis_index("s")
    wid = sid * _NC + cid
    base = wid * _ROWS_PER_TILE

    pltpu.sync_copy(tgt_hbm.at[pl.ds(base, _ROWS_PER_TILE)], tgt_v)
    pltpu.sync_copy(rew_hbm.at[pl.ds(base, _ROWS_PER_TILE)], rew_v)

    def gather_chunk(j0, _):
        t16 = tgt_v[pl.ds(pl.multiple_of(j0 * _L, 8), _L)]
        i0 = base + j0 * _L
        copies = []
        for k in range(_L):
            cp = pltpu.make_async_copy(
                pt_hbm.at[t16[k], pl.ds(i0, _L)],
                vals_v.at[pl.ds(pl.multiple_of((j0 * _L + k) * _L, 8), _L)],
                sem)
            cp.start()
            copies.append(cp)
        for cp in copies:
            cp.wait()
        return ()

    lax.fori_loop(0, _CHUNKS, gather_chunk, ())

    lanes = lax.iota(jnp.int32, _L)
    acc = jnp.zeros((_L,), jnp.float32)
    for j0 in range(_CHUNKS):
        r16 = rew_v[pl.ds(j0 * _L, _L)]
        for k in range(_L):
            g = vals_v[pl.ds((j0 * _L + k) * _L, _L)]
            acc = acc + g * jnp.where(lanes == k, r16, jnp.float32(0))

    buf_v[pl.ds(0, _L)] = acc
    pltpu.sync_copy(buf_v.at[pl.ds(0, _L)],
                    shared.at[pl.ds((cid * _NS + sid) * _L, _L)])
    plsc.subcore_barrier()

    @pl.when(sid == 0)
    def _():
        pltpu.sync_copy(shared.at[pl.ds(cid * _NS * _L, _NS * _L)], sums_v)
        tot = jnp.zeros((_L,), jnp.float32)
        for r in range(_NS):
            tot = tot + sums_v[pl.ds(r * _L, _L)]
        buf_v[pl.ds(0, _L)] = -tot
        pltpu.sync_copy(buf_v.at[pl.ds(0, _L)],
                        out_hbm.at[pl.ds(cid * _L, _L)])


@jax.jit
def _ganloss(prob_t, target_i32, reward):
    mesh = plsc.VectorSubcoreMesh(core_axis_name="c", subcore_axis_name="s")
    run = pl.kernel(
        _ganloss_body,
        out_type=jax.ShapeDtypeStruct((_NC * _L,), jnp.float32),
        mesh=mesh,
        compiler_params=pltpu.CompilerParams(needs_layout_passes=False),
        scratch_types=[
            pltpu.VMEM((_ROWS_PER_TILE,), jnp.int32),
            pltpu.VMEM((_ROWS_PER_TILE,), jnp.float32),
            pltpu.VMEM((_ROWS_PER_TILE * _L,), jnp.float32),
            pltpu.VMEM((_ROWS_PER_TILE,), jnp.float32),
            pltpu.VMEM((_NS * _L,), jnp.float32),
            pltpu.VMEM_SHARED((_NW * _L,), jnp.float32),
            pltpu.SemaphoreType.DMA,
        ],
    )
    out = run(prob_t, target_i32, reward)
    return jnp.sum(out)


def kernel(prob, target, reward):
    return _ganloss(prob.T, target.astype(jnp.int32), reward)


# speedup vs baseline: 146.9770x; 1.2019x over previous
import jax
import jax.numpy as jnp
from jax import lax
from jax.experimental import pallas as pl
from jax.experimental.pallas import tpu as pltpu
from jax.experimental.pallas import tpu_sc as plsc

_N_ROWS = 4096
_N_COLS = 100000
_NC = 2
_NS = 16
_L = 16
_NW = _NC * _NS
_ROWS_PER_TILE = _N_ROWS // _NW
_CHUNKS = _ROWS_PER_TILE // _L


def _ganloss_body(pt_hbm, tgt_hbm, rew_hbm, out_hbm,
                  tgt_v, rew_v, vals_v, buf_v, sem):
    cid = lax.axis_index("c")
    sid = lax.axis_index("s")
    wid = sid * _NC + cid
    base = wid * _ROWS_PER_TILE

    pltpu.sync_copy(tgt_hbm.at[pl.ds(base, _ROWS_PER_TILE)], tgt_v)

    def gather_chunk(j0, _):
        t16 = tgt_v[pl.ds(pl.multiple_of(j0 * _L, 8), _L)]
        i0 = base + j0 * _L
        for k in range(_L):
            pltpu.make_async_copy(
                pt_hbm.at[t16[k], pl.ds(i0, _L)],
                vals_v.at[pl.ds(pl.multiple_of((j0 * _L + k) * _L, 8), _L)],
                sem).start()
        return ()

    lax.fori_loop(0, _CHUNKS, gather_chunk, ())

    pltpu.sync_copy(rew_hbm.at[pl.ds(base, _ROWS_PER_TILE)], rew_v)

    pltpu.make_async_copy(
        pt_hbm.at[0, pl.ds(0, _ROWS_PER_TILE * _L)], vals_v, sem).wait()

    lanes = lax.iota(jnp.int32, _L)
    acc = jnp.zeros((_L,), jnp.float32)
    for j0 in range(_CHUNKS):
        r16 = rew_v[pl.ds(j0 * _L, _L)]
        for k in range(_L):
            g = vals_v[pl.ds((j0 * _L + k) * _L, _L)]
            acc = acc + g * jnp.where(lanes == k, r16, jnp.float32(0))

    buf_v[...] = -acc
    pltpu.sync_copy(buf_v, out_hbm.at[pl.ds(wid * _L, _L)])


@jax.jit
def _ganloss(prob_t, target_i32, reward):
    mesh = plsc.VectorSubcoreMesh(core_axis_name="c", subcore_axis_name="s")
    run = pl.kernel(
        _ganloss_body,
        out_type=jax.ShapeDtypeStruct((_NW * _L,), jnp.float32),
        mesh=mesh,
        compiler_params=pltpu.CompilerParams(needs_layout_passes=False),
        scratch_types=[
            pltpu.VMEM((_ROWS_PER_TILE,), jnp.int32),
            pltpu.VMEM((_ROWS_PER_TILE,), jnp.float32),
            pltpu.VMEM((_ROWS_PER_TILE * _L,), jnp.float32),
            pltpu.VMEM((_L,), jnp.float32),
            pltpu.SemaphoreType.DMA,
        ],
    )
    out = run(prob_t, target_i32, reward)
    return jnp.sum(out)


def kernel(prob, target, reward):
    return _ganloss(prob.T, target.astype(jnp.int32), reward)


# speedup vs baseline: 155.9777x; 1.0612x over previous
import jax
import jax.numpy as jnp
from jax import lax
from jax.experimental import pallas as pl
from jax.experimental.pallas import tpu as pltpu
from jax.experimental.pallas import tpu_sc as plsc

_N_ROWS = 4096
_N_COLS = 100000
_NC = 2
_NS = 16
_L = 16
_NW = _NC * _NS
_ROWS_PER_TILE = _N_ROWS // _NW
_CHUNKS = _ROWS_PER_TILE // _L


def _ganloss_body(pf_hbm, tgt_hbm, rew_hbm, out_hbm,
                  tgt_v, idx_v, rew_v, vals_v, buf_v, sem):
    cid = lax.axis_index("c")
    sid = lax.axis_index("s")
    wid = sid * _NC + cid
    base = wid * _ROWS_PER_TILE

    pltpu.sync_copy(tgt_hbm.at[pl.ds(base, _ROWS_PER_TILE)], tgt_v)

    lanes = lax.iota(jnp.int32, _L)
    for j0 in range(_CHUNKS):
        c = tgt_v[pl.ds(j0 * _L, _L)]
        i = (base + j0 * _L) + lanes
        ih = lax.shift_right_logical(i, 7)
        il = jnp.bitwise_and(i, jnp.int32(127))
        ch = lax.shift_right_logical(c, 3)
        cl = jnp.bitwise_and(c, jnp.int32(7))
        idx_v[pl.ds(j0 * _L, _L)] = (ch * 32 + ih) * 1024 + cl * 128 + il

    cp = pltpu.make_async_copy(pf_hbm.at[idx_v], vals_v, sem)
    cp.start()
    pltpu.sync_copy(rew_hbm.at[pl.ds(base, _ROWS_PER_TILE)], rew_v)
    cp.wait()

    acc = jnp.zeros((_L,), jnp.float32)
    for j0 in range(_CHUNKS):
        acc = acc + vals_v[pl.ds(j0 * _L, _L)] * rew_v[pl.ds(j0 * _L, _L)]

    buf_v[...] = -acc
    pltpu.sync_copy(buf_v, out_hbm.at[pl.ds(wid * _L, _L)])


@jax.jit
def _ganloss(prob, target_i32, reward):
    pf = jnp.transpose(
        prob.reshape(32, 128, 12500, 8), (2, 0, 3, 1)).reshape(-1)
    mesh = plsc.VectorSubcoreMesh(core_axis_name="c", subcore_axis_name="s")
    run = pl.kernel(
        _ganloss_body,
        out_type=jax.ShapeDtypeStruct((_NW * _L,), jnp.float32),
        mesh=mesh,
        compiler_params=pltpu.CompilerParams(needs_layout_passes=False),
        scratch_types=[
            pltpu.VMEM((_ROWS_PER_TILE,), jnp.int32),
            pltpu.VMEM((_ROWS_PER_TILE,), jnp.int32),
            pltpu.VMEM((_ROWS_PER_TILE,), jnp.float32),
            pltpu.VMEM((_ROWS_PER_TILE,), jnp.float32),
            pltpu.VMEM((_L,), jnp.float32),
            pltpu.SemaphoreType.DMA,
        ],
    )
    out = run(pf, target_i32, reward)
    return jnp.sum(out)


def kernel(prob, target, reward):
    return _ganloss(prob, target.astype(jnp.int32), reward)
